# Initial kernel scaffold; baseline (speedup 1.0000x reference)
import jax
import jax.numpy as jnp
from jax.experimental import pallas as pl


def kernel(semantic_code, acoustic_codes, table):
    raise NotImplementedError("write your pallas kernel here")


# speedup vs baseline: 24.6124x; 24.6124x over previous
import functools

import jax
import jax.numpy as jnp
from jax import lax
from jax.experimental import pallas as pl
from jax.experimental.pallas import tpu as pltpu
from jax.experimental.pallas import tpu_sc as plsc

B = 16384
D = 3072
N_AC = 36
AC_SLOT = 23
AC_BASE = 8196
AC_ROWS = N_AC * AC_SLOT
AC_PAD = 832

_NC = 2
_NS = 16
_NW = _NC * _NS
_CH = 16
_B_PER_W = B // _NW
_NCH = _B_PER_W // _CH


def _sc_gather(idx3, table):
    mesh = plsc.VectorSubcoreMesh(core_axis_name="c", subcore_axis_name="s")

    @functools.partial(
        pl.kernel,
        mesh=mesh,
        out_type=jax.ShapeDtypeStruct((B, D), jnp.float32),
        scratch_types=[
            pltpu.VMEM((_NCH, _CH), jnp.int32),
            pltpu.VMEM((2, _CH, D), jnp.float32),
            pltpu.SemaphoreType.DMA,
            pltpu.SemaphoreType.DMA,
        ],
    )
    def k(idx_hbm, table_hbm, out_hbm, idx_v, buf_v, gsem, osem):
        wid = lax.axis_index("s") * _NC + lax.axis_index("c")
        base = wid * _B_PER_W
        pltpu.sync_copy(idx_hbm.at[wid], idx_v)
        pltpu.async_copy(table_hbm.at[idx_v.at[0]], buf_v.at[0], gsem)

        def body(g, carry):
            slot = lax.rem(g, 2)
            nxt = 1 - slot

            @pl.when(g >= 1)
            def _():
                pltpu.make_async_copy(
                    buf_v.at[nxt], out_hbm.at[pl.ds(base + (g - 1) * _CH, _CH)], osem
                ).wait()

            @pl.when(g + 1 < _NCH)
            def _():
                pltpu.async_copy(table_hbm.at[idx_v.at[g + 1]], buf_v.at[nxt], gsem)

            pltpu.make_async_copy(table_hbm.at[idx_v.at[g]], buf_v.at[slot], gsem).wait()
            pltpu.async_copy(buf_v.at[slot], out_hbm.at[pl.ds(base + g * _CH, _CH)], osem)
            return carry

        lax.fori_loop(0, _NCH, body, 0)
        pltpu.make_async_copy(
            buf_v.at[(_NCH - 1) % 2],
            out_hbm.at[pl.ds(base + (_NCH - 1) * _CH, _CH)],
            osem,
        ).wait()

    return k(idx3, table)


_TB = 256


def _combine_body(codes_ref, s_ref, tac_ref, o_ref):
    codes = codes_ref[...].astype(jnp.float32)
    i_of = lax.broadcasted_iota(jnp.int32, (N_AC, AC_PAD), 0)
    j_of = lax.broadcasted_iota(jnp.int32, (N_AC, AC_PAD), 1)
    rep_mat = (i_of == j_of // AC_SLOT).astype(jnp.float32)
    rep = jnp.dot(codes, rep_mat, preferred_element_type=jnp.float32)
    m = (lax.broadcasted_iota(jnp.int32, (_TB, AC_PAD), 1) % AC_SLOT)
    oh = (rep == m.astype(jnp.float32)).astype(jnp.bfloat16)
    ac = jnp.dot(oh, tac_ref[...], preferred_element_type=jnp.float32)
    o_ref[...] = s_ref[...] + ac


def _tc_combine(codes, s, tac):
    return pl.pallas_call(
        _combine_body,
        grid=(B // _TB,),
        in_specs=[
            pl.BlockSpec((_TB, N_AC), lambda i: (i, 0)),
            pl.BlockSpec((_TB, D), lambda i: (i, 0)),
            pl.BlockSpec((AC_PAD, D), lambda i: (0, 0)),
        ],
        out_specs=pl.BlockSpec((_TB, D), lambda i: (i, 0)),
        out_shape=jax.ShapeDtypeStruct((B, D), jnp.float32),
        compiler_params=pltpu.CompilerParams(
            dimension_semantics=("arbitrary",),
        ),
    )(codes, s, tac)


def kernel(semantic_code, acoustic_codes, table):
    sem_idx = (semantic_code.reshape(B).astype(jnp.int32) + 2).reshape(
        _NW, _NCH, _CH
    )
    s = _sc_gather(sem_idx, table)
    tac = jnp.concatenate(
        [
            table[AC_BASE : AC_BASE + AC_ROWS],
            jnp.zeros((AC_PAD - AC_ROWS, D), jnp.float32),
        ]
    ).astype(jnp.bfloat16)
    out = _tc_combine(acoustic_codes.astype(jnp.int32), s, tac)
    return out.reshape(B, 1, D)


# speedup vs baseline: 33.7840x; 1.3726x over previous
import functools

import jax
import jax.numpy as jnp
from jax import lax
from jax.experimental import pallas as pl
from jax.experimental.pallas import tpu as pltpu
from jax.experimental.pallas import tpu_sc as plsc

B = 16384
D = 3072
N_AC = 36
AC_SLOT = 23
AC_BASE = 8196
AC_ROWS = N_AC * AC_SLOT
AC_PAD = 832

_NC = 2
_NS = 16
_NW = _NC * _NS
_CH = 16
_B_PER_W = B // _NW
_NCH = _B_PER_W // _CH


def _sc_gather(idx3, table):
    mesh = plsc.VectorSubcoreMesh(core_axis_name="c", subcore_axis_name="s")

    @functools.partial(
        pl.kernel,
        mesh=mesh,
        out_type=jax.ShapeDtypeStruct((B, D), jnp.float32),
        scratch_types=[
            pltpu.VMEM((_NCH, _CH), jnp.int32),
            pltpu.VMEM((2, _CH, D), jnp.float32),
            pltpu.SemaphoreType.DMA,
            pltpu.SemaphoreType.DMA,
        ],
    )
    def k(idx_hbm, table_hbm, out_hbm, idx_v, buf_v, gsem, osem):
        wid = lax.axis_index("s") * _NC + lax.axis_index("c")
        base = wid * _B_PER_W
        pltpu.sync_copy(idx_hbm.at[wid], idx_v)
        pltpu.async_copy(table_hbm.at[idx_v.at[0]], buf_v.at[0], gsem)

        def body(g, carry):
            slot = lax.rem(g, 2)
            nxt = 1 - slot

            @pl.when(g >= 1)
            def _():
                pltpu.make_async_copy(
                    buf_v.at[nxt], out_hbm.at[pl.ds(base + (g - 1) * _CH, _CH)], osem
                ).wait()

            @pl.when(g + 1 < _NCH)
            def _():
                pltpu.async_copy(table_hbm.at[idx_v.at[g + 1]], buf_v.at[nxt], gsem)

            pltpu.make_async_copy(table_hbm.at[idx_v.at[g]], buf_v.at[slot], gsem).wait()
            pltpu.async_copy(buf_v.at[slot], out_hbm.at[pl.ds(base + g * _CH, _CH)], osem)
            return carry

        lax.fori_loop(0, _NCH, body, 0)
        pltpu.make_async_copy(
            buf_v.at[(_NCH - 1) % 2],
            out_hbm.at[pl.ds(base + (_NCH - 1) * _CH, _CH)],
            osem,
        ).wait()

    return k(idx3, table)


_TB = 256


def _combine_body(codes_ref, s_ref, tac_ref, o_ref):
    codes = codes_ref[...].astype(jnp.float32)
    s = s_ref[...].astype(jnp.float32)
    i_of = lax.broadcasted_iota(jnp.int32, (N_AC, AC_PAD), 0)
    j_of = lax.broadcasted_iota(jnp.int32, (N_AC, AC_PAD), 1)
    rep_mat = (i_of == j_of // AC_SLOT).astype(jnp.float32)
    rep = jnp.dot(codes, rep_mat, preferred_element_type=jnp.float32)
    m = (lax.broadcasted_iota(jnp.int32, (_TB, AC_PAD), 1) % AC_SLOT)
    oh = (rep == m.astype(jnp.float32)).astype(jnp.bfloat16)
    ac = jnp.dot(oh, tac_ref[...], preferred_element_type=jnp.float32)
    o_ref[...] = (s + ac)[:, None, :]


def _tc_combine(codes, s, tac):
    return pl.pallas_call(
        _combine_body,
        grid=(B // _TB,),
        in_specs=[
            pl.BlockSpec((_TB, N_AC), lambda i: (i, 0)),
            pl.BlockSpec((_TB, D), lambda i: (i, 0)),
            pl.BlockSpec((AC_PAD, D), lambda i: (0, 0)),
        ],
        out_specs=pl.BlockSpec((_TB, 1, D), lambda i: (i, 0, 0)),
        out_shape=jax.ShapeDtypeStruct((B, 1, D), jnp.float32),
        compiler_params=pltpu.CompilerParams(
            dimension_semantics=("arbitrary",),
        ),
    )(codes, s, tac)


def kernel(semantic_code, acoustic_codes, table):
    sem_idx = (semantic_code.reshape(B).astype(jnp.int32) + 2).reshape(
        _NW, _NCH, _CH
    )
    s = _sc_gather(sem_idx, table)
    tac = jnp.concatenate(
        [
            table[AC_BASE : AC_BASE + AC_ROWS],
            jnp.zeros((AC_PAD - AC_ROWS, D), jnp.float32),
        ]
    ).astype(jnp.bfloat16)
    return _tc_combine(acoustic_codes.astype(jnp.int32), s, tac)
